# baseline (speedup 1.0000x reference)
import functools

import jax
import jax.numpy as jnp
from jax import lax
from jax.experimental import pallas as pl
from jax.experimental.pallas import tpu as pltpu
from jax.experimental.pallas import tpu_sc as plsc

BATCH = 16384
EMBED = 32
ROWS = 1000000
RBLK = 32768
QBLK = RBLK // 4
NPACK = (ROWS + RBLK - 1) // RBLK
PROWS = NPACK * QBLK
CHUNK = 128


def _repack_body(src, dst):
    eye = (lax.broadcasted_iota(jnp.int32, (128, 128), 0)
           == lax.broadcasted_iota(jnp.int32, (128, 128), 1)
           ).astype(jnp.float32)
    x = src[...]
    xr = jnp.concatenate([x[:, a * QBLK:(a + 1) * QBLK] for a in range(4)],
                         axis=0)
    dst[...] = lax.dot_general(xr, eye, (((0,), (0,)), ((), ())),
                               preferred_element_type=jnp.float32)


def _tc_repack(tab_t):
    return pl.pallas_call(
        _repack_body,
        grid=(NPACK,),
        in_specs=[pl.BlockSpec((EMBED, RBLK), lambda g: (0, g))],
        out_specs=pl.BlockSpec((QBLK, 128), lambda g: (g, 0)),
        out_shape=jax.ShapeDtypeStruct((PROWS, 128), jnp.float32),
    )(tab_t)


def _sc_gather(idx, tab4):
    info = plsc.get_sparse_core_info()
    nw = info.num_cores * info.num_subcores
    b_per_w = BATCH // nw
    nchunk = b_per_w // CHUNK
    mesh = plsc.VectorSubcoreMesh(core_axis_name="c", subcore_axis_name="s")

    @functools.partial(
        pl.kernel,
        mesh=mesh,
        out_type=jax.ShapeDtypeStruct((BATCH, 128), jnp.float32),
        scratch_types=[
            pltpu.VMEM((b_per_w,), jnp.int32),
            pltpu.VMEM((2, CHUNK, 128), jnp.float32),
            pltpu.SemaphoreType.DMA,
        ],
        compiler_params=pltpu.CompilerParams(use_tc_tiling_on_sc=True),
    )
    def gather(idx_hbm, tab_hbm, out_hbm, vidx, rows, sem):
        wid = lax.axis_index("s") * info.num_cores + lax.axis_index("c")
        base = wid * b_per_w
        pltpu.sync_copy(idx_hbm.at[pl.ds(base, b_per_w)], vidx)
        pend = [None] * nchunk
        for j in range(nchunk):
            sl = pl.ds(j * CHUNK, CHUNK)
            pend[j] = pltpu.async_copy(tab_hbm.at[vidx.at[sl]],
                                       rows.at[j % 2], sem)
            if j > 0:
                pend[j - 1].wait()
                out_sl = pl.ds(base + (j - 1) * CHUNK, CHUNK)
                pltpu.sync_copy(rows.at[(j - 1) % 2], out_hbm.at[out_sl])
        pend[nchunk - 1].wait()
        out_sl = pl.ds(base + (nchunk - 1) * CHUNK, CHUNK)
        pltpu.sync_copy(rows.at[(nchunk - 1) % 2], out_hbm.at[out_sl])

    return gather(idx, tab4)


def _mlp_body(u4, i4, us, isel, w1a, w1b, b1, w2, b2, w3, b3, wo, bo, o):
    quarter = lax.broadcasted_iota(jnp.int32, u4.shape, 1) >> 5
    um = jnp.where(quarter == (us[...] // QBLK) % 4, u4[...], 0.0)
    im = jnp.where(quarter == (isel[...] // QBLK) % 4, i4[...], 0.0)
    h = um @ w1a[...] + im @ w1b[...] + b1[...]
    h = jnp.maximum(h, 0.0)
    h = jnp.maximum(h @ w2[...] + b2[...], 0.0)
    h = jnp.maximum(h @ w3[...] + b3[...], 0.0)
    z = jnp.sum(h * wo[...], axis=1, keepdims=True) + bo[...]
    o[...] = 1.0 / (1.0 + jnp.exp(-z))


def _tc_mlp(u4, i4, uids, iids, W1, b1, W2, b2, W3, b3, Wout, bout):
    blk = 4096
    grid = (BATCH // blk,)
    w1a = jnp.tile(W1[:EMBED], (4, 1))
    w1b = jnp.tile(W1[EMBED:], (4, 1))
    full = lambda shape: pl.BlockSpec(shape, lambda g: (0, 0))
    return pl.pallas_call(
        _mlp_body,
        grid=grid,
        in_specs=[
            pl.BlockSpec((blk, 128), lambda g: (g, 0)),
            pl.BlockSpec((blk, 128), lambda g: (g, 0)),
            pl.BlockSpec((blk, 1), lambda g: (g, 0)),
            pl.BlockSpec((blk, 1), lambda g: (g, 0)),
            full(w1a.shape), full(w1b.shape), full((1, 64)),
            full(W2.shape), full((1, 32)),
            full(W3.shape), full((1, 16)),
            full((1, 16)), full((1, 1)),
        ],
        out_specs=pl.BlockSpec((blk, 1), lambda g: (g, 0)),
        out_shape=jax.ShapeDtypeStruct((BATCH, 1), jnp.float32),
    )(u4, i4, uids.reshape(-1, 1), iids.reshape(-1, 1),
      w1a, w1b, b1.reshape(1, -1), W2, b2.reshape(1, -1),
      W3, b3.reshape(1, -1), Wout.reshape(1, -1), bout.reshape(1, 1))


def kernel(user_ids, item_ids, user_table, item_table,
           W1, b1, W2, b2, W3, b3, Wout, bout):
    uids = user_ids.astype(jnp.int32)
    iids = item_ids.astype(jnp.int32)
    uj = (uids // RBLK) * QBLK + (uids % QBLK)
    ij = (iids // RBLK) * QBLK + (iids % QBLK)
    u4 = _tc_repack(user_table.T)
    ug = _sc_gather(uj, u4)
    i4 = _tc_repack(item_table.T)
    ig = _sc_gather(ij, i4)
    return _tc_mlp(ug, ig, uids, iids, W1, b1, W2, b2, W3, b3, Wout, bout)


# speedup vs baseline: 1.0357x; 1.0357x over previous
import functools

import jax
import jax.numpy as jnp
from jax import lax
from jax.experimental import pallas as pl
from jax.experimental.pallas import tpu as pltpu
from jax.experimental.pallas import tpu_sc as plsc

BATCH = 16384
EMBED = 32
ROWS = 1000000
RBLK = 32768
QBLK = RBLK // 4
NPACK = (ROWS + RBLK - 1) // RBLK
PROWS = NPACK * QBLK
CHUNK = 128


def _repack_body(ut_ref, it_ref, u4_ref, i4_ref):
    eye = (lax.broadcasted_iota(jnp.int32, (128, 128), 0)
           == lax.broadcasted_iota(jnp.int32, (128, 128), 1)
           ).astype(jnp.float32)
    for src, dst in ((ut_ref, u4_ref), (it_ref, i4_ref)):
        x = src[...]
        xr = jnp.concatenate([x[:, a * QBLK:(a + 1) * QBLK] for a in range(4)],
                             axis=0)
        dst[...] = lax.dot_general(xr, eye, (((0,), (0,)), ((), ())),
                                   preferred_element_type=jnp.float32)


def _tc_repack(ut_t, it_t):
    return pl.pallas_call(
        _repack_body,
        grid=(NPACK,),
        in_specs=[pl.BlockSpec((EMBED, RBLK), lambda g: (0, g)),
                  pl.BlockSpec((EMBED, RBLK), lambda g: (0, g))],
        out_specs=[pl.BlockSpec((QBLK, 128), lambda g: (g, 0)),
                   pl.BlockSpec((QBLK, 128), lambda g: (g, 0))],
        out_shape=[jax.ShapeDtypeStruct((PROWS, 128), jnp.float32)] * 2,
    )(ut_t, it_t)


def _sc_gather(uj, ij, user_t4, item_t4):
    info = plsc.get_sparse_core_info()
    nw = info.num_cores * info.num_subcores
    b_per_w = BATCH // nw
    nchunk = b_per_w // CHUNK
    mesh = plsc.VectorSubcoreMesh(core_axis_name="c", subcore_axis_name="s")

    @functools.partial(
        pl.kernel,
        mesh=mesh,
        out_type=[
            jax.ShapeDtypeStruct((BATCH, 128), jnp.float32),
            jax.ShapeDtypeStruct((BATCH, 128), jnp.float32),
        ],
        scratch_types=[
            pltpu.VMEM((b_per_w,), jnp.int32),
            pltpu.VMEM((b_per_w,), jnp.int32),
            pltpu.VMEM((2, CHUNK, 128), jnp.float32),
            pltpu.VMEM((2, CHUNK, 128), jnp.float32),
            pltpu.SemaphoreType.DMA,
        ],
        compiler_params=pltpu.CompilerParams(use_tc_tiling_on_sc=True),
    )
    def gather(uid_hbm, iid_hbm, ut_hbm, it_hbm, u_out, i_out,
               uidx, iidx, urows, irows, sem):
        wid = lax.axis_index("s") * info.num_cores + lax.axis_index("c")
        base = wid * b_per_w
        pltpu.sync_copy(uid_hbm.at[pl.ds(base, b_per_w)], uidx)
        pltpu.sync_copy(iid_hbm.at[pl.ds(base, b_per_w)], iidx)
        pend = [None] * nchunk
        for j in range(nchunk):
            sl = pl.ds(j * CHUNK, CHUNK)
            pend[j] = (
                pltpu.async_copy(ut_hbm.at[uidx.at[sl]], urows.at[j % 2], sem),
                pltpu.async_copy(it_hbm.at[iidx.at[sl]], irows.at[j % 2], sem),
            )
            if j > 0:
                for c in pend[j - 1]:
                    c.wait()
                out_sl = pl.ds(base + (j - 1) * CHUNK, CHUNK)
                pltpu.sync_copy(urows.at[(j - 1) % 2], u_out.at[out_sl])
                pltpu.sync_copy(irows.at[(j - 1) % 2], i_out.at[out_sl])
        for c in pend[nchunk - 1]:
            c.wait()
        out_sl = pl.ds(base + (nchunk - 1) * CHUNK, CHUNK)
        pltpu.sync_copy(urows.at[(nchunk - 1) % 2], u_out.at[out_sl])
        pltpu.sync_copy(irows.at[(nchunk - 1) % 2], i_out.at[out_sl])

    return gather(uj, ij, user_t4, item_t4)


def _mlp_body(u4, i4, us, isel, w1a, w1b, b1, w2, b2, w3, b3, wo, bo, o):
    quarter = lax.broadcasted_iota(jnp.int32, u4.shape, 1) >> 5
    um = jnp.where(quarter == (us[...] // QBLK) % 4, u4[...], 0.0)
    im = jnp.where(quarter == (isel[...] // QBLK) % 4, i4[...], 0.0)
    h = um @ w1a[...] + im @ w1b[...] + b1[...]
    h = jnp.maximum(h, 0.0)
    h = jnp.maximum(h @ w2[...] + b2[...], 0.0)
    h = jnp.maximum(h @ w3[...] + b3[...], 0.0)
    z = jnp.sum(h * wo[...], axis=1, keepdims=True) + bo[...]
    o[...] = 1.0 / (1.0 + jnp.exp(-z))


def _tc_mlp(u4, i4, uids, iids, W1, b1, W2, b2, W3, b3, Wout, bout):
    blk = 4096
    grid = (BATCH // blk,)
    w1a = jnp.tile(W1[:EMBED], (4, 1))
    w1b = jnp.tile(W1[EMBED:], (4, 1))
    full = lambda shape: pl.BlockSpec(shape, lambda g: (0, 0))
    return pl.pallas_call(
        _mlp_body,
        grid=grid,
        in_specs=[
            pl.BlockSpec((blk, 128), lambda g: (g, 0)),
            pl.BlockSpec((blk, 128), lambda g: (g, 0)),
            pl.BlockSpec((blk, 1), lambda g: (g, 0)),
            pl.BlockSpec((blk, 1), lambda g: (g, 0)),
            full(w1a.shape), full(w1b.shape), full((1, 64)),
            full(W2.shape), full((1, 32)),
            full(W3.shape), full((1, 16)),
            full((1, 16)), full((1, 1)),
        ],
        out_specs=pl.BlockSpec((blk, 1), lambda g: (g, 0)),
        out_shape=jax.ShapeDtypeStruct((BATCH, 1), jnp.float32),
    )(u4, i4, uids.reshape(-1, 1), iids.reshape(-1, 1),
      w1a, w1b, b1.reshape(1, -1), W2, b2.reshape(1, -1),
      W3, b3.reshape(1, -1), Wout.reshape(1, -1), bout.reshape(1, 1))


def kernel(user_ids, item_ids, user_table, item_table,
           W1, b1, W2, b2, W3, b3, Wout, bout):
    uids = user_ids.astype(jnp.int32)
    iids = item_ids.astype(jnp.int32)
    u4, i4 = _tc_repack(user_table.T, item_table.T)
    uj = (uids // RBLK) * QBLK + (uids % QBLK)
    ij = (iids // RBLK) * QBLK + (iids % QBLK)
    ug, ig = _sc_gather(uj, ij, u4, i4)
    return _tc_mlp(ug, ig, uids, iids, W1, b1, W2, b2, W3, b3, Wout, bout)
